# Initial kernel scaffold; baseline (speedup 1.0000x reference)
import jax
import jax.numpy as jnp
from jax.experimental import pallas as pl


def kernel(mes_update, yv, cu_seqlens):
    raise NotImplementedError("write your pallas kernel here")


# speedup vs baseline: 1.8100x; 1.8100x over previous
import jax
import jax.numpy as jnp
from jax import lax
from jax.experimental import pallas as pl
from jax.experimental.pallas import tpu as pltpu
from jax.experimental.pallas import tpu_sc as plsc

TOTAL = 8192
D = 1024
NB = 8
NC = 2
NS = 16
NW = NC * NS
RPW = TOTAL // NW
CH = 32
NCHUNK = RPW // CH
L = 16
G = 2
GW = D // G
GS = GW // L
STRIP = NB * D // NS


def _lane_extract(vec, lane, i):
    return jnp.sum(jnp.where(lane == i, vec, 0))


def _phase1_body(mes_hbm, yv_hbm, cu_hbm, zeros_hbm,
                 p0_out, p1_out, yvcas_out,
                 buf0, buf1, acc,
                 cuv, yvbuf, rbuf0, rbuf1, stage,
                 sem0, sem1, semyv, semz, semr0, semr1):
    c = lax.axis_index("c")
    s = lax.axis_index("s")
    wid = s * NC + c
    base = wid * RPW

    zcp = pltpu.async_copy(zeros_hbm, acc, semz)
    pltpu.sync_copy(cu_hbm, cuv)

    @pl.when(jnp.logical_and(c == 0, s == 1))
    def _():
        pltpu.async_copy(yv_hbm.at[cuv.at[pl.ds(0, NB)]], yvbuf, semyv).wait()
        pltpu.sync_copy(yvbuf, yvcas_out)

    cu_val = cuv[...]
    lane = lax.iota(jnp.int32, L)
    cub = [_lane_extract(cu_val, lane, b) for b in range(1, NB + 1)]

    def seg_of(pos):
        seg = jnp.int32(0)
        for b in range(NB - 1):
            seg = seg + (cub[b] <= pos).astype(jnp.int32)
        return seg

    bufs = (buf0, buf1)
    sems = (sem0, sem1)
    copies = [None, None]
    copies[0] = pltpu.async_copy(
        mes_hbm.at[pl.ds(base, CH)], buf0, sem0)
    if NCHUNK > 1:
        copies[1] = pltpu.async_copy(
            mes_hbm.at[pl.ds(base + CH, CH)], buf1, sem1)

    zcp.wait()
    for j in range(NCHUNK):
        p = j % 2
        copies[p].wait()
        buf = bufs[p]
        cstart = base + j * CH

        sfirst = seg_of(cstart)
        slast = seg_of(cstart + (CH - 1))

        def b_body(b, _, buf=buf, cstart=cstart):
            cu_lo = _lane_extract(cu_val, lane, b)
            cu_hi = _lane_extract(cu_val, lane, b + 1)
            lo = jnp.clip(cu_lo - cstart, 0, CH)
            hi = jnp.clip(cu_hi - cstart, 0, CH)
            for g in range(G):
                def r_body(r, carry, buf=buf, g=g):
                    return tuple(
                        carry[k] + buf[r, pl.ds(g * GW + k * L, L)]
                        for k in range(GS)
                    )
                carry = lax.fori_loop(
                    lo, hi, r_body,
                    tuple(jnp.zeros((L,), jnp.float32) for _ in range(GS)))
                for k in range(GS):
                    sl = pl.ds(b * D + g * GW + k * L, L)
                    acc[sl] = acc[sl] + carry[k]
            return 0

        lax.fori_loop(sfirst, slast + 1, b_body, 0)

        nxt = j + 2
        if nxt < NCHUNK:
            copies[p] = pltpu.async_copy(
                mes_hbm.at[pl.ds(base + nxt * CH, CH)], bufs[p], sems[p])

    pltpu.sync_copy(acc, stage.at[s])
    plsc.subcore_barrier()

    rbufs = (rbuf0, rbuf1)
    rsems = (semr0, semr1)
    rcp = [None, None]
    rcp[0] = pltpu.async_copy(stage.at[0, pl.ds(s * STRIP, STRIP)],
                              rbuf0, semr0)
    rcp[1] = pltpu.async_copy(stage.at[1, pl.ds(s * STRIP, STRIP)],
                              rbuf1, semr1)
    total = [jnp.zeros((L,), jnp.float32) for _ in range(STRIP // L)]
    for i in range(NS):
        p = i % 2
        rcp[p].wait()
        for k in range(STRIP // L):
            total[k] = total[k] + rbufs[p][pl.ds(k * L, L)]
        nxt = i + 2
        if nxt < NS:
            rcp[p] = pltpu.async_copy(
                stage.at[nxt, pl.ds(s * STRIP, STRIP)], rbufs[p], rsems[p])
    for k in range(STRIP // L):
        rbuf0[pl.ds(k * L, L)] = total[k]

    @pl.when(c == 0)
    def _():
        pltpu.sync_copy(rbuf0, p0_out.at[pl.ds(s * STRIP, STRIP)])

    @pl.when(c == 1)
    def _():
        pltpu.sync_copy(rbuf0, p1_out.at[pl.ds(s * STRIP, STRIP)])


def _phase2_body(p0_hbm, p1_hbm, cu_hbm, out_hbm,
                 av, bv, ov, cuv):
    c = lax.axis_index("c")
    s = lax.axis_index("s")
    wid = s * NC + c
    r = wid % NB
    q = wid // NB
    span = D // (NW // NB)
    off = r * D + q * span

    pltpu.sync_copy(cu_hbm, cuv)
    pltpu.sync_copy(p0_hbm.at[pl.ds(off, span)], av)
    pltpu.sync_copy(p1_hbm.at[pl.ds(off, span)], bv)

    cu_val = cuv[...]
    lane = lax.iota(jnp.int32, L)
    hi = _lane_extract(cu_val, lane, r + 1)
    lo = _lane_extract(cu_val, lane, r)
    cnt = (hi - lo).astype(jnp.float32)

    for i in range(span // L):
        sl = pl.ds(i * L, L)
        ov[sl] = (av[sl] + bv[sl]) / cnt

    pltpu.sync_copy(ov, out_hbm.at[pl.ds(off, span)])


@jax.jit
def _run(mes_update, yv, cu_pad, zeros):
    mesh = plsc.VectorSubcoreMesh(core_axis_name="c", subcore_axis_name="s")

    params = pltpu.CompilerParams(needs_layout_passes=False)
    phase1 = pl.kernel(
        _phase1_body,
        mesh=mesh,
        compiler_params=params,
        out_type=[
            jax.ShapeDtypeStruct((NB * D,), jnp.float32),
            jax.ShapeDtypeStruct((NB * D,), jnp.float32),
            jax.ShapeDtypeStruct((NB, D), jnp.float32),
        ],
        scratch_types=[
            pltpu.VMEM((CH, D), jnp.float32),
            pltpu.VMEM((CH, D), jnp.float32),
            pltpu.VMEM((NB * D,), jnp.float32),
            pltpu.VMEM((L,), jnp.int32),
            pltpu.VMEM((NB, D), jnp.float32),
            pltpu.VMEM((STRIP,), jnp.float32),
            pltpu.VMEM((STRIP,), jnp.float32),
            pltpu.VMEM_SHARED((NS, NB * D), jnp.float32),
            pltpu.SemaphoreType.DMA,
            pltpu.SemaphoreType.DMA,
            pltpu.SemaphoreType.DMA,
            pltpu.SemaphoreType.DMA,
            pltpu.SemaphoreType.DMA,
            pltpu.SemaphoreType.DMA,
        ],
    )
    p0, p1, yv_cas = phase1(mes_update, yv, cu_pad, zeros)

    phase2 = pl.kernel(
        _phase2_body,
        mesh=mesh,
        compiler_params=params,
        out_type=jax.ShapeDtypeStruct((NB * D,), jnp.float32),
        scratch_types=[
            pltpu.VMEM((D // (NW // NB),), jnp.float32),
            pltpu.VMEM((D // (NW // NB),), jnp.float32),
            pltpu.VMEM((D // (NW // NB),), jnp.float32),
            pltpu.VMEM((L,), jnp.int32),
        ],
    )
    mean_flat = phase2(p0, p1, cu_pad)
    return mean_flat.reshape(NB, D), yv_cas


def kernel(mes_update, yv, cu_seqlens):
    cu_pad = jnp.pad(cu_seqlens.astype(jnp.int32), (0, L - NB - 1),
                     mode="edge")
    zeros = jnp.zeros((NB * D,), jnp.float32)
    return _run(mes_update, yv, cu_pad, zeros)


# speedup vs baseline: 2.3067x; 1.2744x over previous
import jax
import jax.numpy as jnp
from jax import lax
from jax.experimental import pallas as pl
from jax.experimental.pallas import tpu as pltpu
from jax.experimental.pallas import tpu_sc as plsc

TOTAL = 8192
D = 1024
NB = 8
NC = 2
NS = 16
NW = NC * NS
TC_ROWS = 6144
TCR = 512
SC_ROWS = TOTAL - TC_ROWS
RPW = SC_ROWS // NW
CH = 32
NCHUNK = RPW // CH
L = 16
G = 2
GW = D // G
GS = GW // L
STRIP = NB * D // NS


def _lane_extract(vec, lane, i):
    return jnp.sum(jnp.where(lane == i, vec, 0))


def _phase1_body(mes_hbm, yv_hbm, cu_hbm, zeros_hbm,
                 p0_out, p1_out, yvcas_out,
                 buf0, buf1, acc,
                 cuv, yvbuf, rbuf0, rbuf1, stage,
                 sem0, sem1, semyv, semz, semr0, semr1):
    c = lax.axis_index("c")
    s = lax.axis_index("s")
    wid = s * NC + c
    base = TC_ROWS + wid * RPW

    zcp = pltpu.async_copy(zeros_hbm, acc, semz)
    pltpu.sync_copy(cu_hbm, cuv)

    @pl.when(jnp.logical_and(c == 0, s == 1))
    def _():
        pltpu.async_copy(yv_hbm.at[cuv.at[pl.ds(0, NB)]], yvbuf, semyv).wait()
        pltpu.sync_copy(yvbuf, yvcas_out)

    cu_val = cuv[...]
    lane = lax.iota(jnp.int32, L)
    cub = [_lane_extract(cu_val, lane, b) for b in range(1, NB + 1)]

    def seg_of(pos):
        seg = jnp.int32(0)
        for b in range(NB - 1):
            seg = seg + (cub[b] <= pos).astype(jnp.int32)
        return seg

    bufs = (buf0, buf1)
    sems = (sem0, sem1)
    copies = [None, None]
    copies[0] = pltpu.async_copy(
        mes_hbm.at[pl.ds(base, CH)], buf0, sem0)
    if NCHUNK > 1:
        copies[1] = pltpu.async_copy(
            mes_hbm.at[pl.ds(base + CH, CH)], buf1, sem1)

    zcp.wait()
    for j in range(NCHUNK):
        p = j % 2
        copies[p].wait()
        buf = bufs[p]
        cstart = base + j * CH

        sfirst = seg_of(cstart)
        slast = seg_of(cstart + (CH - 1))

        def b_body(b, _, buf=buf, cstart=cstart):
            cu_lo = _lane_extract(cu_val, lane, b)
            cu_hi = _lane_extract(cu_val, lane, b + 1)
            lo = jnp.clip(cu_lo - cstart, 0, CH)
            hi = jnp.clip(cu_hi - cstart, 0, CH)
            for g in range(G):
                def r_body(r, carry, buf=buf, g=g):
                    return tuple(
                        carry[k] + buf[r, pl.ds(g * GW + k * L, L)]
                        for k in range(GS)
                    )
                carry = lax.fori_loop(
                    lo, hi, r_body,
                    tuple(jnp.zeros((L,), jnp.float32) for _ in range(GS)))
                for k in range(GS):
                    sl = pl.ds(b * D + g * GW + k * L, L)
                    acc[sl] = acc[sl] + carry[k]
            return 0

        lax.fori_loop(sfirst, slast + 1, b_body, 0)

        nxt = j + 2
        if nxt < NCHUNK:
            copies[p] = pltpu.async_copy(
                mes_hbm.at[pl.ds(base + nxt * CH, CH)], bufs[p], sems[p])

    pltpu.sync_copy(acc, stage.at[s])
    plsc.subcore_barrier()

    rbufs = (rbuf0, rbuf1)
    rsems = (semr0, semr1)
    rcp = [None, None]
    rcp[0] = pltpu.async_copy(stage.at[0, pl.ds(s * STRIP, STRIP)],
                              rbuf0, semr0)
    rcp[1] = pltpu.async_copy(stage.at[1, pl.ds(s * STRIP, STRIP)],
                              rbuf1, semr1)
    total = [jnp.zeros((L,), jnp.float32) for _ in range(STRIP // L)]
    for i in range(NS):
        p = i % 2
        rcp[p].wait()
        for k in range(STRIP // L):
            total[k] = total[k] + rbufs[p][pl.ds(k * L, L)]
        nxt = i + 2
        if nxt < NS:
            rcp[p] = pltpu.async_copy(
                stage.at[nxt, pl.ds(s * STRIP, STRIP)], rbufs[p], rsems[p])
    for k in range(STRIP // L):
        rbuf0[pl.ds(k * L, L)] = total[k]

    @pl.when(c == 0)
    def _():
        pltpu.sync_copy(rbuf0, p0_out.at[pl.ds(s * STRIP, STRIP)])

    @pl.when(c == 1)
    def _():
        pltpu.sync_copy(rbuf0, p1_out.at[pl.ds(s * STRIP, STRIP)])


def _tc_sum_body(cu_ref, mes_ref, out_ref):
    j = pl.program_id(0)

    @pl.when(j == 0)
    def _():
        out_ref[...] = jnp.zeros_like(out_ref)

    rows = lax.broadcasted_iota(jnp.int32, (NB, TCR), 1) + j * TCR
    lo = jnp.stack([cu_ref[b] for b in range(NB)])[:, None]
    hi = jnp.stack([cu_ref[b + 1] for b in range(NB)])[:, None]
    m = jnp.logical_and(lo <= rows, rows < hi).astype(jnp.float32)
    out_ref[...] += lax.dot(m, mes_ref[...],
                            preferred_element_type=jnp.float32)


def _phase2_body(p0_hbm, p1_hbm, p2_hbm, cu_hbm, out_hbm,
                 av, bv, cv, ov, cuv):
    c = lax.axis_index("c")
    s = lax.axis_index("s")
    wid = s * NC + c
    r = wid % NB
    q = wid // NB
    span = D // (NW // NB)
    off = r * D + q * span

    pltpu.sync_copy(cu_hbm, cuv)
    pltpu.sync_copy(p0_hbm.at[pl.ds(off, span)], av)
    pltpu.sync_copy(p1_hbm.at[pl.ds(off, span)], bv)
    pltpu.sync_copy(p2_hbm.at[pl.ds(off, span)], cv)

    cu_val = cuv[...]
    lane = lax.iota(jnp.int32, L)
    hi = _lane_extract(cu_val, lane, r + 1)
    lo = _lane_extract(cu_val, lane, r)
    cnt = (hi - lo).astype(jnp.float32)

    for i in range(span // L):
        sl = pl.ds(i * L, L)
        ov[sl] = (av[sl] + bv[sl] + cv[sl]) / cnt

    pltpu.sync_copy(ov, out_hbm.at[pl.ds(off, span)])


@jax.jit
def _run(mes_update, yv, cu_pad, zeros):
    mesh = plsc.VectorSubcoreMesh(core_axis_name="c", subcore_axis_name="s")

    params = pltpu.CompilerParams(needs_layout_passes=False)
    phase1 = pl.kernel(
        _phase1_body,
        mesh=mesh,
        compiler_params=params,
        out_type=[
            jax.ShapeDtypeStruct((NB * D,), jnp.float32),
            jax.ShapeDtypeStruct((NB * D,), jnp.float32),
            jax.ShapeDtypeStruct((NB, D), jnp.float32),
        ],
        scratch_types=[
            pltpu.VMEM((CH, D), jnp.float32),
            pltpu.VMEM((CH, D), jnp.float32),
            pltpu.VMEM((NB * D,), jnp.float32),
            pltpu.VMEM((L,), jnp.int32),
            pltpu.VMEM((NB, D), jnp.float32),
            pltpu.VMEM((STRIP,), jnp.float32),
            pltpu.VMEM((STRIP,), jnp.float32),
            pltpu.VMEM_SHARED((NS, NB * D), jnp.float32),
            pltpu.SemaphoreType.DMA,
            pltpu.SemaphoreType.DMA,
            pltpu.SemaphoreType.DMA,
            pltpu.SemaphoreType.DMA,
            pltpu.SemaphoreType.DMA,
            pltpu.SemaphoreType.DMA,
        ],
    )
    p0, p1, yv_cas = phase1(mes_update, yv, cu_pad, zeros)

    ptc = pl.pallas_call(
        _tc_sum_body,
        grid=(TC_ROWS // TCR,),
        in_specs=[
            pl.BlockSpec(memory_space=pltpu.SMEM),
            pl.BlockSpec((TCR, D), lambda j: (j, 0)),
        ],
        out_specs=pl.BlockSpec((NB, D), lambda j: (0, 0)),
        out_shape=jax.ShapeDtypeStruct((NB, D), jnp.float32),
    )(cu_pad, mes_update)

    phase2 = pl.kernel(
        _phase2_body,
        mesh=mesh,
        compiler_params=params,
        out_type=jax.ShapeDtypeStruct((NB * D,), jnp.float32),
        scratch_types=[
            pltpu.VMEM((D // (NW // NB),), jnp.float32),
            pltpu.VMEM((D // (NW // NB),), jnp.float32),
            pltpu.VMEM((D // (NW // NB),), jnp.float32),
            pltpu.VMEM((D // (NW // NB),), jnp.float32),
            pltpu.VMEM((L,), jnp.int32),
        ],
    )
    mean_flat = phase2(p0, p1, ptc.reshape(-1), cu_pad)
    return mean_flat.reshape(NB, D), yv_cas


def kernel(mes_update, yv, cu_seqlens):
    cu_pad = jnp.pad(cu_seqlens.astype(jnp.int32), (0, L - NB - 1),
                     mode="edge")
    zeros = jnp.zeros((NB * D,), jnp.float32)
    return _run(mes_update, yv, cu_pad, zeros)


# speedup vs baseline: 2.8807x; 1.2488x over previous
import jax
import jax.numpy as jnp
from jax import lax
from jax.experimental import pallas as pl
from jax.experimental.pallas import tpu as pltpu
from jax.experimental.pallas import tpu_sc as plsc

TOTAL = 8192
D = 1024
NB = 8
NC = 2
NS = 16
NW = NC * NS
TC_ROWS = 6144
TCR = 512
SC_ROWS = TOTAL - TC_ROWS
RPW = SC_ROWS // NW
CH = RPW // 2
NCHUNK = RPW // CH
L = 16
G = 2
GW = D // G
GS = GW // L
STRIP = NB * D // NS


def _lane_extract(vec, lane, i):
    return jnp.sum(jnp.where(lane == i, vec, 0))


def _sc_body(mes_hbm, yv_hbm, cu_hbm,
             p0_out, p1_out, yvcas_out,
             buf0, buf1, acc, cuv, yvbuf, rbuf0, rbuf1, stage,
             sem0, sem1, semyv, semr0, semr1):
    c = lax.axis_index("c")
    s = lax.axis_index("s")
    wid = s * NC + c
    base = TC_ROWS + wid * RPW

    bufs = (buf0, buf1)
    sems = (sem0, sem1)
    copies = [None, None]
    copies[0] = pltpu.async_copy(
        mes_hbm.at[pl.ds(base, CH)], buf0, sem0)
    if NCHUNK > 1:
        copies[1] = pltpu.async_copy(
            mes_hbm.at[pl.ds(base + CH, CH)], buf1, sem1)

    pltpu.sync_copy(cu_hbm, cuv.at[pl.ds(0, NB + 1)])

    yv_worker = jnp.logical_and(c == 0, s == 1)
    yv_copy = [None]

    @pl.when(yv_worker)
    def _():
        yv_copy[0] = pltpu.async_copy(
            yv_hbm.at[cuv.at[pl.ds(0, NB)]], yvbuf, semyv)

    def z_body(i, _):
        for k in range(8):
            acc[pl.ds(i * 128 + k * L, L)] = jnp.zeros((L,), jnp.float32)
        return 0
    lax.fori_loop(0, NB * D // 128, z_body, 0)

    cu_val = cuv[...]
    lane = lax.iota(jnp.int32, L)
    cub = [_lane_extract(cu_val, lane, b) for b in range(1, NB + 1)]

    def seg_of(pos):
        seg = jnp.int32(0)
        for b in range(NB - 1):
            seg = seg + (cub[b] <= pos).astype(jnp.int32)
        return seg

    for j in range(NCHUNK):
        p = j % 2
        copies[p].wait()
        buf = bufs[p]
        cstart = base + j * CH

        sfirst = seg_of(cstart)
        slast = seg_of(cstart + (CH - 1))

        def b_body(b, _, buf=buf, cstart=cstart):
            cu_lo = _lane_extract(cu_val, lane, b)
            cu_hi = _lane_extract(cu_val, lane, b + 1)
            lo = jnp.clip(cu_lo - cstart, 0, CH)
            hi = jnp.clip(cu_hi - cstart, 0, CH)
            for g in range(G):
                def r_body(r, carry, buf=buf, g=g):
                    return tuple(
                        carry[k] + buf[r, pl.ds(g * GW + k * L, L)]
                        for k in range(GS)
                    )
                carry = lax.fori_loop(
                    lo, hi, r_body,
                    tuple(jnp.zeros((L,), jnp.float32) for _ in range(GS)))
                for k in range(GS):
                    sl = pl.ds(b * D + g * GW + k * L, L)
                    acc[sl] = acc[sl] + carry[k]
            return 0

        lax.fori_loop(sfirst, slast + 1, b_body, 0)

    pltpu.sync_copy(acc, stage.at[s])
    plsc.subcore_barrier()

    rbufs = (rbuf0, rbuf1)
    rsems = (semr0, semr1)
    rcp = [None, None]
    rcp[0] = pltpu.async_copy(stage.at[0, pl.ds(s * STRIP, STRIP)],
                              rbuf0, semr0)
    rcp[1] = pltpu.async_copy(stage.at[1, pl.ds(s * STRIP, STRIP)],
                              rbuf1, semr1)
    total = [jnp.zeros((L,), jnp.float32) for _ in range(STRIP // L)]
    for i in range(NS):
        p = i % 2
        rcp[p].wait()
        for k in range(STRIP // L):
            total[k] = total[k] + rbufs[p][pl.ds(k * L, L)]
        nxt = i + 2
        if nxt < NS:
            rcp[p] = pltpu.async_copy(
                stage.at[nxt, pl.ds(s * STRIP, STRIP)], rbufs[p], rsems[p])
    for k in range(STRIP // L):
        rbuf0[pl.ds(k * L, L)] = total[k]

    row = s // 2
    half = (s % 2) * STRIP

    @pl.when(c == 0)
    def _():
        pltpu.sync_copy(rbuf0, p0_out.at[row, pl.ds(half, STRIP)])

    @pl.when(c == 1)
    def _():
        pltpu.sync_copy(rbuf0, p1_out.at[row, pl.ds(half, STRIP)])

    @pl.when(yv_worker)
    def _():
        yv_copy[0].wait()
        pltpu.sync_copy(yvbuf, yvcas_out)


def _tc_sum_body(cu_ref, mes_ref, out_ref):
    j = pl.program_id(0)

    @pl.when(j == 0)
    def _():
        out_ref[...] = jnp.zeros_like(out_ref)

    rows = lax.broadcasted_iota(jnp.int32, (NB, TCR), 1) + j * TCR
    lo = jnp.stack([cu_ref[b] for b in range(NB)])[:, None]
    hi = jnp.stack([cu_ref[b + 1] for b in range(NB)])[:, None]
    m = jnp.logical_and(lo <= rows, rows < hi).astype(jnp.float32)
    out_ref[...] += lax.dot(m, mes_ref[...],
                            preferred_element_type=jnp.float32)


def _tc_combine_body(cu_ref, p0_ref, p1_ref, ptc_ref, out_ref):
    cnt = jnp.stack([cu_ref[b + 1] - cu_ref[b] for b in range(NB)])
    cntf = cnt[:, None].astype(jnp.float32)
    out_ref[...] = (p0_ref[...] + p1_ref[...] + ptc_ref[...]) / cntf


@jax.jit
def _run(mes_update, yv, cu_seqlens):
    mesh = plsc.VectorSubcoreMesh(core_axis_name="c", subcore_axis_name="s")

    params = pltpu.CompilerParams(needs_layout_passes=False)
    sc_kernel = pl.kernel(
        _sc_body,
        mesh=mesh,
        compiler_params=params,
        out_type=[
            jax.ShapeDtypeStruct((NB, D), jnp.float32),
            jax.ShapeDtypeStruct((NB, D), jnp.float32),
            jax.ShapeDtypeStruct((NB, D), jnp.float32),
        ],
        scratch_types=[
            pltpu.VMEM((CH, D), jnp.float32),
            pltpu.VMEM((CH, D), jnp.float32),
            pltpu.VMEM((NB * D,), jnp.float32),
            pltpu.VMEM((L,), jnp.int32),
            pltpu.VMEM((NB, D), jnp.float32),
            pltpu.VMEM((STRIP,), jnp.float32),
            pltpu.VMEM((STRIP,), jnp.float32),
            pltpu.VMEM_SHARED((NS, NB * D), jnp.float32),
            pltpu.SemaphoreType.DMA,
            pltpu.SemaphoreType.DMA,
            pltpu.SemaphoreType.DMA,
            pltpu.SemaphoreType.DMA,
            pltpu.SemaphoreType.DMA,
        ],
    )
    p0, p1, yv_cas = sc_kernel(mes_update, yv, cu_seqlens)

    ptc = pl.pallas_call(
        _tc_sum_body,
        grid=(TC_ROWS // TCR,),
        in_specs=[
            pl.BlockSpec(memory_space=pltpu.SMEM),
            pl.BlockSpec((TCR, D), lambda j: (j, 0)),
        ],
        out_specs=pl.BlockSpec((NB, D), lambda j: (0, 0)),
        out_shape=jax.ShapeDtypeStruct((NB, D), jnp.float32),
    )(cu_seqlens, mes_update)

    cas_mean = pl.pallas_call(
        _tc_combine_body,
        in_specs=[
            pl.BlockSpec(memory_space=pltpu.SMEM),
            pl.BlockSpec((NB, D)),
            pl.BlockSpec((NB, D)),
            pl.BlockSpec((NB, D)),
        ],
        out_specs=pl.BlockSpec((NB, D)),
        out_shape=jax.ShapeDtypeStruct((NB, D), jnp.float32),
    )(cu_seqlens, p0, p1, ptc)

    return cas_mean, yv_cas


def kernel(mes_update, yv, cu_seqlens):
    return _run(mes_update, yv, cu_seqlens.astype(jnp.int32))


# speedup vs baseline: 3.0906x; 1.0729x over previous
import jax
import jax.numpy as jnp
from jax import lax
from jax.experimental import pallas as pl
from jax.experimental.pallas import tpu as pltpu
from jax.experimental.pallas import tpu_sc as plsc

TOTAL = 8192
D = 1024
NB = 8
NC = 2
NS = 16
NW = NC * NS
TC_ROWS = 6144
TCR = 1024
SC_ROWS = TOTAL - TC_ROWS
RPW = SC_ROWS // NW
CH = RPW // 2
NCHUNK = RPW // CH
L = 16
G = 2
GW = D // G
GS = GW // L
STRIP = NB * D // NS


def _lane_extract(vec, lane, i):
    return jnp.sum(jnp.where(lane == i, vec, 0))


def _sc_body(mes_hbm, yv_hbm, cu_hbm,
             p0_out, p1_out, yvcas_out,
             buf0, buf1, acc, cuv, yvbuf, rbuf0, rbuf2d, stage,
             sem0, sem1, semyv, semr):
    c = lax.axis_index("c")
    s = lax.axis_index("s")
    wid = s * NC + c
    base = TC_ROWS + wid * RPW

    bufs = (buf0, buf1)
    sems = (sem0, sem1)
    copies = [None, None]
    copies[0] = pltpu.async_copy(
        mes_hbm.at[pl.ds(base, CH)], buf0, sem0)
    if NCHUNK > 1:
        copies[1] = pltpu.async_copy(
            mes_hbm.at[pl.ds(base + CH, CH)], buf1, sem1)

    pltpu.sync_copy(cu_hbm, cuv.at[pl.ds(0, NB + 1)])

    yv_worker = jnp.logical_and(c == 0, s == 1)
    yv_copy = [None]

    @pl.when(yv_worker)
    def _():
        yv_copy[0] = pltpu.async_copy(
            yv_hbm.at[cuv.at[pl.ds(0, NB)]], yvbuf, semyv)

    def z_body(i, _):
        for k in range(8):
            acc[pl.ds(i * 128 + k * L, L)] = jnp.zeros((L,), jnp.float32)
        return 0
    lax.fori_loop(0, NB * D // 128, z_body, 0)

    cu_val = cuv[...]
    lane = lax.iota(jnp.int32, L)
    cub = [_lane_extract(cu_val, lane, b) for b in range(1, NB + 1)]

    def seg_of(pos):
        seg = jnp.int32(0)
        for b in range(NB - 1):
            seg = seg + (cub[b] <= pos).astype(jnp.int32)
        return seg

    for j in range(NCHUNK):
        p = j % 2
        copies[p].wait()
        buf = bufs[p]
        cstart = base + j * CH

        sfirst = seg_of(cstart)
        slast = seg_of(cstart + (CH - 1))

        def b_body(b, _, buf=buf, cstart=cstart):
            cu_lo = _lane_extract(cu_val, lane, b)
            cu_hi = _lane_extract(cu_val, lane, b + 1)
            lo = jnp.clip(cu_lo - cstart, 0, CH)
            hi = jnp.clip(cu_hi - cstart, 0, CH)
            for g in range(G):
                def r_body(r, carry, buf=buf, g=g):
                    return tuple(
                        carry[k] + buf[r, pl.ds(g * GW + k * L, L)]
                        for k in range(GS)
                    )
                carry = lax.fori_loop(
                    lo, hi, r_body,
                    tuple(jnp.zeros((L,), jnp.float32) for _ in range(GS)))
                for k in range(GS):
                    sl = pl.ds(b * D + g * GW + k * L, L)
                    acc[sl] = acc[sl] + carry[k]
            return 0

        lax.fori_loop(sfirst, slast + 1, b_body, 0)

    pltpu.sync_copy(acc, stage.at[s])
    plsc.subcore_barrier()

    def issue_body(i, _):
        pltpu.async_copy(stage.at[i, pl.ds(s * STRIP, STRIP)],
                         rbuf2d.at[i], semr)
        return 0
    lax.fori_loop(0, NS, issue_body, 0)

    def drain_body(i, _):
        pltpu.make_async_copy(stage.at[0, pl.ds(s * STRIP, STRIP)],
                              rbuf2d.at[0], semr).wait()
        return 0
    lax.fori_loop(0, NS, drain_body, 0)

    def red_body(i, carry):
        return tuple(
            carry[k] + rbuf2d[i, pl.ds(k * L, L)]
            for k in range(STRIP // L)
        )
    total = lax.fori_loop(
        0, NS, red_body,
        tuple(jnp.zeros((L,), jnp.float32) for _ in range(STRIP // L)))
    for k in range(STRIP // L):
        rbuf0[pl.ds(k * L, L)] = total[k]

    row = s // 2
    half = (s % 2) * STRIP

    @pl.when(c == 0)
    def _():
        pltpu.sync_copy(rbuf0, p0_out.at[row, pl.ds(half, STRIP)])

    @pl.when(c == 1)
    def _():
        pltpu.sync_copy(rbuf0, p1_out.at[row, pl.ds(half, STRIP)])

    @pl.when(yv_worker)
    def _():
        yv_copy[0].wait()
        pltpu.sync_copy(yvbuf, yvcas_out)


def _tc_sum_body(cu_ref, mes_ref, out_ref):
    j = pl.program_id(0)

    @pl.when(j == 0)
    def _():
        out_ref[...] = jnp.zeros_like(out_ref)

    rows = lax.broadcasted_iota(jnp.int32, (NB, TCR), 1) + j * TCR
    lo = jnp.stack([cu_ref[b] for b in range(NB)])[:, None]
    hi = jnp.stack([cu_ref[b + 1] for b in range(NB)])[:, None]
    m = jnp.logical_and(lo <= rows, rows < hi).astype(jnp.float32)
    out_ref[...] += lax.dot(m, mes_ref[...],
                            preferred_element_type=jnp.float32)


def _tc_combine_body(cu_ref, p0_ref, p1_ref, ptc_ref, out_ref):
    cnt = jnp.stack([cu_ref[b + 1] - cu_ref[b] for b in range(NB)])
    cntf = cnt[:, None].astype(jnp.float32)
    out_ref[...] = (p0_ref[...] + p1_ref[...] + ptc_ref[...]) / cntf


@jax.jit
def _run(mes_update, yv, cu_seqlens):
    mesh = plsc.VectorSubcoreMesh(core_axis_name="c", subcore_axis_name="s")

    params = pltpu.CompilerParams(needs_layout_passes=False)
    sc_kernel = pl.kernel(
        _sc_body,
        mesh=mesh,
        compiler_params=params,
        out_type=[
            jax.ShapeDtypeStruct((NB, D), jnp.float32),
            jax.ShapeDtypeStruct((NB, D), jnp.float32),
            jax.ShapeDtypeStruct((NB, D), jnp.float32),
        ],
        scratch_types=[
            pltpu.VMEM((CH, D), jnp.float32),
            pltpu.VMEM((CH, D), jnp.float32),
            pltpu.VMEM((NB * D,), jnp.float32),
            pltpu.VMEM((L,), jnp.int32),
            pltpu.VMEM((NB, D), jnp.float32),
            pltpu.VMEM((STRIP,), jnp.float32),
            pltpu.VMEM((NS, STRIP), jnp.float32),
            pltpu.VMEM_SHARED((NS, NB * D), jnp.float32),
            pltpu.SemaphoreType.DMA,
            pltpu.SemaphoreType.DMA,
            pltpu.SemaphoreType.DMA,
            pltpu.SemaphoreType.DMA,
        ],
    )
    ptc = pl.pallas_call(
        _tc_sum_body,
        grid=(TC_ROWS // TCR,),
        in_specs=[
            pl.BlockSpec(memory_space=pltpu.SMEM),
            pl.BlockSpec((TCR, D), lambda j: (j, 0)),
        ],
        out_specs=pl.BlockSpec((NB, D), lambda j: (0, 0)),
        out_shape=jax.ShapeDtypeStruct((NB, D), jnp.float32),
    )(cu_seqlens, mes_update)

    p0, p1, yv_cas = sc_kernel(mes_update, yv, cu_seqlens)

    cas_mean = pl.pallas_call(
        _tc_combine_body,
        in_specs=[
            pl.BlockSpec(memory_space=pltpu.SMEM),
            pl.BlockSpec((NB, D)),
            pl.BlockSpec((NB, D)),
            pl.BlockSpec((NB, D)),
        ],
        out_specs=pl.BlockSpec((NB, D)),
        out_shape=jax.ShapeDtypeStruct((NB, D), jnp.float32),
    )(cu_seqlens, p0, p1, ptc)

    return cas_mean, yv_cas


def kernel(mes_update, yv, cu_seqlens):
    return _run(mes_update, yv, cu_seqlens.astype(jnp.int32))


# speedup vs baseline: 3.4303x; 1.1099x over previous
import jax
import jax.numpy as jnp
from jax import lax
from jax.experimental import pallas as pl
from jax.experimental.pallas import tpu as pltpu
from jax.experimental.pallas import tpu_sc as plsc

TOTAL = 8192
D = 1024
NB = 8
NC = 2
NS = 16
NW = NC * NS
TC_ROWS = 7168
TCR = 1024
SC_ROWS = TOTAL - TC_ROWS
RPW = SC_ROWS // NW
CH = RPW
NCHUNK = RPW // CH
L = 16
G = 2
GW = D // G
GS = GW // L
STRIP = NB * D // NS


def _lane_extract(vec, lane, i):
    return jnp.sum(jnp.where(lane == i, vec, 0))


def _sc_body(mes_hbm, yv_hbm, cu_hbm,
             p0_out, p1_out, yvcas_out,
             buf0, buf1, acc, cuv, yvbuf, rbuf0, rbuf2d, stage,
             sem0, sem1, semyv, semr):
    c = lax.axis_index("c")
    s = lax.axis_index("s")
    wid = s * NC + c
    base = TC_ROWS + wid * RPW

    bufs = (buf0, buf1)
    sems = (sem0, sem1)
    copies = [None, None]
    copies[0] = pltpu.async_copy(
        mes_hbm.at[pl.ds(base, CH)], buf0, sem0)
    if NCHUNK > 1:
        copies[1] = pltpu.async_copy(
            mes_hbm.at[pl.ds(base + CH, CH)], buf1, sem1)

    pltpu.sync_copy(cu_hbm, cuv.at[pl.ds(0, NB + 1)])

    yv_worker = jnp.logical_and(c == 0, s == 1)
    yv_copy = [None]

    @pl.when(yv_worker)
    def _():
        yv_copy[0] = pltpu.async_copy(
            yv_hbm.at[cuv.at[pl.ds(0, NB)]], yvbuf, semyv)

    def z_body(i, _):
        for k in range(8):
            acc[pl.ds(i * 128 + k * L, L)] = jnp.zeros((L,), jnp.float32)
        return 0
    lax.fori_loop(0, NB * D // 128, z_body, 0)

    cu_val = cuv[...]
    lane = lax.iota(jnp.int32, L)
    cub = [_lane_extract(cu_val, lane, b) for b in range(1, NB + 1)]

    def seg_of(pos):
        seg = jnp.int32(0)
        for b in range(NB - 1):
            seg = seg + (cub[b] <= pos).astype(jnp.int32)
        return seg

    for j in range(NCHUNK):
        p = j % 2
        copies[p].wait()
        buf = bufs[p]
        cstart = base + j * CH

        sfirst = seg_of(cstart)
        slast = seg_of(cstart + (CH - 1))

        def b_body(b, _, buf=buf, cstart=cstart):
            cu_lo = _lane_extract(cu_val, lane, b)
            cu_hi = _lane_extract(cu_val, lane, b + 1)
            lo = jnp.clip(cu_lo - cstart, 0, CH)
            hi = jnp.clip(cu_hi - cstart, 0, CH)
            for g in range(G):
                def r_body(r, carry, buf=buf, g=g):
                    return tuple(
                        carry[k] + buf[r, pl.ds(g * GW + k * L, L)]
                        for k in range(GS)
                    )
                carry = lax.fori_loop(
                    lo, hi, r_body,
                    tuple(jnp.zeros((L,), jnp.float32) for _ in range(GS)))
                for k in range(GS):
                    sl = pl.ds(b * D + g * GW + k * L, L)
                    acc[sl] = acc[sl] + carry[k]
            return 0

        lax.fori_loop(sfirst, slast + 1, b_body, 0)

    pltpu.sync_copy(acc, stage.at[s])
    plsc.subcore_barrier()

    def issue_body(i, _):
        pltpu.async_copy(stage.at[i, pl.ds(s * STRIP, STRIP)],
                         rbuf2d.at[i], semr)
        return 0
    lax.fori_loop(0, NS, issue_body, 0)

    def drain_body(i, _):
        pltpu.make_async_copy(stage.at[0, pl.ds(s * STRIP, STRIP)],
                              rbuf2d.at[0], semr).wait()
        return 0
    lax.fori_loop(0, NS, drain_body, 0)

    def red_body(i, carry):
        return tuple(
            carry[k] + rbuf2d[i, pl.ds(k * L, L)]
            for k in range(STRIP // L)
        )
    total = lax.fori_loop(
        0, NS, red_body,
        tuple(jnp.zeros((L,), jnp.float32) for _ in range(STRIP // L)))
    for k in range(STRIP // L):
        rbuf0[pl.ds(k * L, L)] = total[k]

    row = s // 2
    half = (s % 2) * STRIP

    @pl.when(c == 0)
    def _():
        pltpu.sync_copy(rbuf0, p0_out.at[row, pl.ds(half, STRIP)])

    @pl.when(c == 1)
    def _():
        pltpu.sync_copy(rbuf0, p1_out.at[row, pl.ds(half, STRIP)])

    @pl.when(yv_worker)
    def _():
        yv_copy[0].wait()
        pltpu.sync_copy(yvbuf, yvcas_out)


def _tc_sum_body(cu_ref, mes_ref, out_ref):
    j = pl.program_id(0)

    @pl.when(j == 0)
    def _():
        out_ref[...] = jnp.zeros_like(out_ref)

    rows = lax.broadcasted_iota(jnp.int32, (NB, TCR), 1) + j * TCR
    lo = jnp.stack([cu_ref[b] for b in range(NB)])[:, None]
    hi = jnp.stack([cu_ref[b + 1] for b in range(NB)])[:, None]
    m = jnp.logical_and(lo <= rows, rows < hi).astype(jnp.float32)
    out_ref[...] += lax.dot(m, mes_ref[...],
                            preferred_element_type=jnp.float32)


def _tc_combine_body(cu_ref, p0_ref, p1_ref, ptc_ref, out_ref):
    cnt = jnp.stack([cu_ref[b + 1] - cu_ref[b] for b in range(NB)])
    cntf = cnt[:, None].astype(jnp.float32)
    out_ref[...] = (p0_ref[...] + p1_ref[...] + ptc_ref[...]) / cntf


@jax.jit
def _run(mes_update, yv, cu_seqlens):
    mesh = plsc.VectorSubcoreMesh(core_axis_name="c", subcore_axis_name="s")

    params = pltpu.CompilerParams(needs_layout_passes=False)
    sc_kernel = pl.kernel(
        _sc_body,
        mesh=mesh,
        compiler_params=params,
        out_type=[
            jax.ShapeDtypeStruct((NB, D), jnp.float32),
            jax.ShapeDtypeStruct((NB, D), jnp.float32),
            jax.ShapeDtypeStruct((NB, D), jnp.float32),
        ],
        scratch_types=[
            pltpu.VMEM((CH, D), jnp.float32),
            pltpu.VMEM((CH, D), jnp.float32),
            pltpu.VMEM((NB * D,), jnp.float32),
            pltpu.VMEM((L,), jnp.int32),
            pltpu.VMEM((NB, D), jnp.float32),
            pltpu.VMEM((STRIP,), jnp.float32),
            pltpu.VMEM((NS, STRIP), jnp.float32),
            pltpu.VMEM_SHARED((NS, NB * D), jnp.float32),
            pltpu.SemaphoreType.DMA,
            pltpu.SemaphoreType.DMA,
            pltpu.SemaphoreType.DMA,
            pltpu.SemaphoreType.DMA,
        ],
    )
    ptc = pl.pallas_call(
        _tc_sum_body,
        grid=(TC_ROWS // TCR,),
        in_specs=[
            pl.BlockSpec(memory_space=pltpu.SMEM),
            pl.BlockSpec((TCR, D), lambda j: (j, 0)),
        ],
        out_specs=pl.BlockSpec((NB, D), lambda j: (0, 0)),
        out_shape=jax.ShapeDtypeStruct((NB, D), jnp.float32),
    )(cu_seqlens, mes_update)

    p0, p1, yv_cas = sc_kernel(mes_update, yv, cu_seqlens)

    cas_mean = pl.pallas_call(
        _tc_combine_body,
        in_specs=[
            pl.BlockSpec(memory_space=pltpu.SMEM),
            pl.BlockSpec((NB, D)),
            pl.BlockSpec((NB, D)),
            pl.BlockSpec((NB, D)),
        ],
        out_specs=pl.BlockSpec((NB, D)),
        out_shape=jax.ShapeDtypeStruct((NB, D), jnp.float32),
    )(cu_seqlens, p0, p1, ptc)

    return cas_mean, yv_cas


def kernel(mes_update, yv, cu_seqlens):
    return _run(mes_update, yv, cu_seqlens.astype(jnp.int32))
